# Initial kernel scaffold; baseline (speedup 1.0000x reference)
import jax
import jax.numpy as jnp
from jax.experimental import pallas as pl


def kernel(slow, fast, gaze_maps, w_slow_t, w_fast_t, bias_row):
    raise NotImplementedError("write your pallas kernel here")


# speedup vs baseline: 1.0871x; 1.0871x over previous
import jax
import jax.numpy as jnp
from jax.experimental import pallas as pl
from jax.experimental.pallas import tpu as pltpu


def _ipow(x, p):
    result = None
    base = x
    while p > 0:
        if p & 1:
            result = base if result is None else result * base
        p >>= 1
        if p:
            base = base * base
    return result


def _make_pool_body(cs, bug, inv_ls, inv_lf, pow_s):
    def body(slow_ref, fast_ref, gf_ref, gs_ref, sp_ref, fp_ref):
        slowf = slow_ref[0]
        sums = jnp.sum(slowf, axis=-1, keepdims=True)
        g = _ipow(gs_ref[0], pow_s)
        row = slowf[bug:bug + 1, :]
        corr = jnp.sum(row * g, axis=-1, keepdims=True)
        cid = jax.lax.broadcasted_iota(jnp.int32, (cs, 1), 0)
        fixed = jnp.where(cid == bug, corr, sums)
        sp_ref[0, 0, :] = fixed[:, 0] * inv_ls

        fastf = fast_ref[0]
        fp = jnp.sum(fastf * gf_ref[0], axis=-1, keepdims=True)
        fp_ref[0, 0, :] = fp[:, 0] * inv_lf
    return body


def _head_body(xs_ref, xf_ref, ws_ref, wf_ref, b_ref, o_ref):
    logits = (jnp.dot(xs_ref[...], ws_ref[...], preferred_element_type=jnp.float32)
              + jnp.dot(xf_ref[...], wf_ref[...], preferred_element_type=jnp.float32)
              + b_ref[...])
    m = jnp.max(logits, axis=-1, keepdims=True)
    e = jnp.exp(logits - m)
    o_ref[...] = e / jnp.sum(e, axis=-1, keepdims=True)


def kernel(slow, fast, gaze_maps, w_slow_t, w_fast_t, bias_row):
    N, Cs, Ts, H, W = slow.shape
    _, Cf, Tf, _, _ = fast.shape
    alpha = Tf // Ts
    Ls, Lf = Ts * H * W, Tf * H * W
    K = w_slow_t.shape[1]
    bug = Cf - 1

    slow2 = slow.reshape(N, Cs, Ls)
    fast2 = fast.reshape(N, Cf, Lf)
    gaze_f = gaze_maps.reshape(N, 1, Lf)
    gaze_s = gaze_maps[:, ::alpha].reshape(N, 1, Ls)

    slow_pooled, fast_pooled = pl.pallas_call(
        _make_pool_body(Cs, bug, 1.0 / Ls, 1.0 / Lf, Cs),
        out_shape=[
            jax.ShapeDtypeStruct((N, 1, Cs), jnp.float32),
            jax.ShapeDtypeStruct((N, 1, Cf), jnp.float32),
        ],
        grid=(N,),
        in_specs=[
            pl.BlockSpec((1, Cs, Ls), lambda n: (n, 0, 0)),
            pl.BlockSpec((1, Cf, Lf), lambda n: (n, 0, 0)),
            pl.BlockSpec((1, 1, Lf), lambda n: (n, 0, 0)),
            pl.BlockSpec((1, 1, Ls), lambda n: (n, 0, 0)),
        ],
        out_specs=[
            pl.BlockSpec((1, 1, Cs), lambda n: (n, 0, 0)),
            pl.BlockSpec((1, 1, Cf), lambda n: (n, 0, 0)),
        ],
        compiler_params=pltpu.CompilerParams(
            dimension_semantics=("parallel",)),
    )(slow2, fast2, gaze_f, gaze_s)

    return pl.pallas_call(
        _head_body,
        out_shape=jax.ShapeDtypeStruct((N, K), jnp.float32),
        grid=(1,),
        in_specs=[
            pl.BlockSpec((N, Cs), lambda i: (0, 0)),
            pl.BlockSpec((N, Cf), lambda i: (0, 0)),
            pl.BlockSpec((Cs, K), lambda i: (0, 0)),
            pl.BlockSpec((Cf, K), lambda i: (0, 0)),
            pl.BlockSpec((1, K), lambda i: (0, 0)),
        ],
        out_specs=pl.BlockSpec((N, K), lambda i: (0, 0)),
    )(slow_pooled.reshape(N, Cs), fast_pooled.reshape(N, Cf),
      w_slow_t, w_fast_t, bias_row)
